# baseline (speedup 1.0000x reference)
import functools
import jax
import jax.numpy as jnp
from jax import lax
from jax.experimental import pallas as pl
from jax.experimental.pallas import tpu as pltpu
from jax.experimental.pallas import tpu_sc as plsc

_MAX_REL = 32
_S = 512
_D = 768
_NROWS = 2 * _MAX_REL + 1
_NC = 1
_NS = 16
_CBLKS = 16
_CW = _D // _CBLKS
_L = 16
_CV = _CW // _L
_INV = 1.0 / _S


def _rpe_sc_body(table_hbm, out_hbm, tbl_v, out_v):
    wid = lax.axis_index("s")
    c0 = wid * _CW

    pltpu.sync_copy(table_hbm.at[:, pl.ds(c0, _CW)], tbl_v)

    sls = [pl.ds(k * _L, _L) for k in range(_CV)]
    inv = jnp.float32(_INV)

    acc = []
    for k in range(_CV):
        ones = None
        for r in range(1, _MAX_REL + 1):
            t = tbl_v[r, sls[k]]
            ones = t if ones is None else ones + t
        a = ones + tbl_v[0, sls[k]] * jnp.float32(_S - _MAX_REL)
        a = a * inv
        out_v[0, sls[k]] = a
        acc.append(a)

    d_mid = [(tbl_v[_NROWS - 1, sls[k]] - tbl_v[0, sls[k]]) * inv for k in range(_CV)]

    for j in range(1, _S):
        hi = min(j, _MAX_REL) + _MAX_REL
        lo = max(j - (_S - _MAX_REL), 0)
        for k in range(_CV):
            if hi == _NROWS - 1 and lo == 0:
                a = acc[k] + d_mid[k]
            else:
                a = acc[k] + (tbl_v[hi, sls[k]] - tbl_v[lo, sls[k]]) * inv
            out_v[j, sls[k]] = a
            acc[k] = a

    pltpu.sync_copy(out_v, out_hbm.at[:, pl.ds(c0, _CW)])


def kernel(seq_len, table):
    mesh = plsc.VectorSubcoreMesh(
        core_axis_name="c", subcore_axis_name="s", num_cores=_NC, num_subcores=_NS
    )
    rpe = functools.partial(
        pl.kernel,
        out_type=jax.ShapeDtypeStruct((_S, _D), jnp.float32),
        mesh=mesh,
        scratch_types=[
            pltpu.VMEM((_NROWS, _CW), jnp.float32),
            pltpu.VMEM((_S, _CW), jnp.float32),
        ],
        compiler_params=pltpu.CompilerParams(use_tc_tiling_on_sc=False),
    )(_rpe_sc_body)
    return rpe(table)[None, :, :]


# speedup vs baseline: 1.0488x; 1.0488x over previous
import functools
import jax
import jax.numpy as jnp
from jax import lax
from jax.experimental import pallas as pl
from jax.experimental.pallas import tpu as pltpu
from jax.experimental.pallas import tpu_sc as plsc

_MAX_REL = 32
_S = 512
_D = 768
_NROWS = 2 * _MAX_REL + 1
_NC = 2
_NS = 16
_CBLKS = 16
_RBLKS = 2
_CW = _D // _CBLKS
_RH = _S // _RBLKS
_HH = _RH // 2
_L = 16
_CV = _CW // _L
_INV = 1.0 / _S


def _row_weight(r, j):
    if r == 0:
        return max(0, _S - _MAX_REL - j)
    if r == _NROWS - 1:
        return max(0, j - (_MAX_REL - 1))
    v = r - _MAX_REL
    return 1 if (v <= j and v >= j - (_S - 1)) else 0


def _walk(tbl_v, out_v, out_hbm, c0, sem1, sem2, j0):
    sls = [pl.ds(k * _L, _L) for k in range(_CV)]
    inv = jnp.float32(_INV)

    acc = []
    for k in range(_CV):
        ones = None
        for r in range(_NROWS):
            if _row_weight(r, j0) == 1:
                t = tbl_v[r, sls[k]]
                ones = t if ones is None else ones + t
        a = ones
        for r in (0, _NROWS - 1):
            w = _row_weight(r, j0)
            if w > 1:
                a = a + tbl_v[r, sls[k]] * jnp.float32(w)
        a = a * inv
        out_v[0, sls[k]] = a
        acc.append(a)

    d_mid = [(tbl_v[_NROWS - 1, sls[k]] - tbl_v[0, sls[k]]) * inv for k in range(_CV)]

    cp1 = None
    anchor = acc
    s_anchor = 0
    for s in range(1, _RH):
        j = j0 + s
        hi = min(j, _MAX_REL) + _MAX_REL
        lo = max(j - (_S - _MAX_REL), 0)
        if hi == _NROWS - 1 and lo == 0:
            n = jnp.float32(s - s_anchor)
            row = [anchor[k] + n * d_mid[k] for k in range(_CV)]
        else:
            row = [acc[k] + (tbl_v[hi, sls[k]] - tbl_v[lo, sls[k]]) * inv
                   for k in range(_CV)]
            anchor = row
            s_anchor = s
        for k in range(_CV):
            out_v[s, sls[k]] = row[k]
        acc = row
        if s == _HH - 1:
            cp1 = pltpu.async_copy(
                out_v.at[pl.ds(0, _HH)],
                out_hbm.at[pl.ds(j0, _HH), pl.ds(c0, _CW)],
                sem1,
            )

    cp2 = pltpu.async_copy(
        out_v.at[pl.ds(_HH, _HH)],
        out_hbm.at[pl.ds(j0 + _HH, _HH), pl.ds(c0, _CW)],
        sem2,
    )
    cp1.wait()
    cp2.wait()


def _rpe_sc_body(table_hbm, out_hbm, tbl_v, out_v, sem1, sem2):
    wid = lax.axis_index("s") * _NC + lax.axis_index("c")
    cb = wid % _CBLKS
    rb = wid // _CBLKS
    c0 = cb * _CW

    pltpu.sync_copy(table_hbm.at[:, pl.ds(c0, _CW)], tbl_v)

    for blk in range(_RBLKS):
        @pl.when(rb == blk)
        def _():
            _walk(tbl_v, out_v, out_hbm, c0, sem1, sem2, blk * _RH)


def kernel(seq_len, table):
    mesh = plsc.VectorSubcoreMesh(
        core_axis_name="c", subcore_axis_name="s", num_cores=_NC, num_subcores=_NS
    )
    rpe = functools.partial(
        pl.kernel,
        out_type=jax.ShapeDtypeStruct((_S, _D), jnp.float32),
        mesh=mesh,
        scratch_types=[
            pltpu.VMEM((_NROWS, _CW), jnp.float32),
            pltpu.VMEM((_RH, _CW), jnp.float32),
            pltpu.SemaphoreType.DMA,
            pltpu.SemaphoreType.DMA,
        ],
        compiler_params=pltpu.CompilerParams(use_tc_tiling_on_sc=False),
    )(_rpe_sc_body)
    return rpe(table)[None, :, :]


# speedup vs baseline: 1.0559x; 1.0068x over previous
import functools
import jax
import jax.numpy as jnp
from jax import lax
from jax.experimental import pallas as pl
from jax.experimental.pallas import tpu as pltpu
from jax.experimental.pallas import tpu_sc as plsc

_MAX_REL = 32
_S = 512
_D = 768
_NROWS = 2 * _MAX_REL + 1
_NC = 2
_NS = 16
_CBLKS = 16
_RBLKS = 2
_CW = _D // _CBLKS
_RH = _S // _RBLKS
_L = 16
_CV = _CW // _L
_INV = 1.0 / _S


def _row_weight(r, j):
    if r == 0:
        return max(0, _S - _MAX_REL - j)
    if r == _NROWS - 1:
        return max(0, j - (_MAX_REL - 1))
    v = r - _MAX_REL
    return 1 if (v <= j and v >= j - (_S - 1)) else 0


def _tree_sum(terms):
    while len(terms) > 1:
        nxt = [terms[i] + terms[i + 1] for i in range(0, len(terms) - 1, 2)]
        if len(terms) % 2:
            nxt.append(terms[-1])
        terms = nxt
    return terms[0]


def _walk(tbl_v, out_v, out_hbm, c0, sems, j0):
    sls = [pl.ds(k * _L, _L) for k in range(_CV)]
    inv = jnp.float32(_INV)

    acc = []
    for k in range(_CV):
        terms = [tbl_v[r, sls[k]]
                 for r in range(_NROWS) if _row_weight(r, j0) == 1]
        for r in (0, _NROWS - 1):
            w = _row_weight(r, j0)
            if w > 1:
                terms.append(tbl_v[r, sls[k]] * jnp.float32(w))
        a = _tree_sum(terms) * inv
        out_v[0, sls[k]] = a
        acc.append(a)

    d_mid = [(tbl_v[_NROWS - 1, sls[k]] - tbl_v[0, sls[k]]) * inv for k in range(_CV)]

    nq = len(sems)
    qh = _RH // nq
    cps = []
    anchor = acc
    s_anchor = 0
    for s in range(1, _RH):
        j = j0 + s
        hi = min(j, _MAX_REL) + _MAX_REL
        lo = max(j - (_S - _MAX_REL), 0)
        if hi == _NROWS - 1 and lo == 0:
            n = jnp.float32(s - s_anchor)
            row = [anchor[k] + n * d_mid[k] for k in range(_CV)]
        else:
            row = [acc[k] + (tbl_v[hi, sls[k]] - tbl_v[lo, sls[k]]) * inv
                   for k in range(_CV)]
            anchor = row
            s_anchor = s
        for k in range(_CV):
            out_v[s, sls[k]] = row[k]
        acc = row
        if s % qh == qh - 1:
            q = s // qh
            cps.append(pltpu.async_copy(
                out_v.at[pl.ds(q * qh, qh)],
                out_hbm.at[pl.ds(j0 + q * qh, qh), pl.ds(c0, _CW)],
                sems[q],
            ))
    for cp in cps:
        cp.wait()


def _rpe_sc_body(table_hbm, out_hbm, tbl_v, out_v, *sems):
    wid = lax.axis_index("s") * _NC + lax.axis_index("c")
    cb = wid % _CBLKS
    rb = wid // _CBLKS
    c0 = cb * _CW

    pltpu.sync_copy(table_hbm.at[:, pl.ds(c0, _CW)], tbl_v)

    for blk in range(_RBLKS):
        @pl.when(rb == blk)
        def _():
            _walk(tbl_v, out_v, out_hbm, c0, sems, blk * _RH)


def kernel(seq_len, table):
    mesh = plsc.VectorSubcoreMesh(
        core_axis_name="c", subcore_axis_name="s", num_cores=_NC, num_subcores=_NS
    )
    rpe = functools.partial(
        pl.kernel,
        out_type=jax.ShapeDtypeStruct((_S, _D), jnp.float32),
        mesh=mesh,
        scratch_types=[
            pltpu.VMEM((_NROWS, _CW), jnp.float32),
            pltpu.VMEM((_RH, _CW), jnp.float32),
            pltpu.SemaphoreType.DMA,
            pltpu.SemaphoreType.DMA,
            pltpu.SemaphoreType.DMA,
            pltpu.SemaphoreType.DMA,
        ],
        compiler_params=pltpu.CompilerParams(use_tc_tiling_on_sc=False),
    )(_rpe_sc_body)
    return rpe(table)[None, :, :]
